# baseline (speedup 1.0000x reference)
import functools

import jax
import jax.numpy as jnp
from jax import lax
from jax.experimental import pallas as pl
from jax.experimental.pallas import tpu as pltpu
from jax.experimental.pallas import tpu_sc as plsc

N = 10000
E = 320000
D = 128
DP = 136
NC = 2
NS = 16
NW = NC * NS
EPW = E // NW
CK = 80
NCHUNK = EPW // CK
NP = 10240
RPT = NP // NS


def _sc_agg_body(xp_hbm, pk_hbm, agg_out,
                 pk_t, gb0, gb1, gb2, srcb, dstb, accum,
                 semg0, semg1, semg2, sems0, sems1, sems2):
    c = lax.axis_index("c")
    s = lax.axis_index("s")
    wid = s * NC + c

    zeros16 = jnp.zeros((16,), jnp.float32)
    gbufs = (gb0, gb1, gb2)
    semgs = (semg0, semg1, semg2)
    semss = (sems0, sems1, sems2)

    def zero_gb0(i, carry):
        for k in range(8):
            gb0[i, pl.ds(k * 16, 16)] = zeros16
        gb0[i, pl.ds(DP - 16, 16)] = zeros16
        return carry
    lax.fori_loop(0, CK, zero_gb0, 0)

    base = s * RPT
    for k in range(RPT // CK):
        pltpu.sync_copy(gb0, accum.at[pl.ds(base + k * CK, CK)])

    pltpu.sync_copy(pk_hbm.at[pl.ds(wid * EPW, EPW)], pk_t)

    plsc.subcore_barrier()

    mask = jnp.full((16,), 0xFFFF, jnp.int32)

    def unpack(t, slot):
        for k in range(CK // 16):
            pkv = pk_t[pl.ds(t * CK + k * 16, 16)]
            srcb[slot, pl.ds(k * 16, 16)] = jnp.bitwise_and(pkv, mask)
            dstb[slot, pl.ds(k * 16, 16)] = lax.shift_right_logical(pkv, 16)

    def g_issue(t, b):
        pltpu.async_copy(xp_hbm.at[srcb.at[b]], gbufs[b], semgs[b])

    def g_wait(b):
        pltpu.make_async_copy(xp_hbm.at[srcb.at[b]], gbufs[b],
                              semgs[b]).wait()

    def s_issue(b):
        pltpu.async_copy(gbufs[b], accum.at[dstb.at[b]], semss[b], add=True)

    def s_wait(b):
        pltpu.make_async_copy(gbufs[b], accum.at[dstb.at[b]],
                              semss[b]).wait()

    unpack(0, 0)
    g_issue(0, 0)
    unpack(1, 1)
    g_issue(1, 1)
    g_wait(0); s_issue(0); unpack(2, 2); g_issue(2, 2)
    g_wait(1); s_issue(1); s_wait(0); unpack(3, 0); g_issue(3, 0)

    def triple(j, carry):
        t = 3 * j + 2
        for p, b in ((0, 2), (1, 0), (2, 1)):
            g_wait(b)
            s_issue(b)
            s_wait((b + 2) % 3)
            unpack(t + p + 2, (b + 2) % 3)
            g_issue(t + p + 2, (b + 2) % 3)
        return carry
    lax.fori_loop(0, 40, triple, 0)

    g_wait(2); s_issue(2); s_wait(1); unpack(124, 1); g_issue(124, 1)
    g_wait(0); s_issue(0); s_wait(2)
    g_wait(1); pltpu.sync_copy(gbufs[1], accum.at[dstb.at[1]], add=True)
    s_wait(0)

    plsc.subcore_barrier()

    pltpu.sync_copy(accum.at[pl.ds(base, RPT)],
                    agg_out.at[c, pl.ds(base, RPT)])


_sc_agg = functools.partial(
    pl.kernel,
    out_type=jax.ShapeDtypeStruct((NC, NP, DP), jnp.float32),
    mesh=plsc.VectorSubcoreMesh(
        core_axis_name="c", subcore_axis_name="s",
        num_cores=NC, num_subcores=NS),
    scratch_types=[
        pltpu.VMEM((EPW,), jnp.int32),
        pltpu.VMEM((CK, DP), jnp.float32),
        pltpu.VMEM((CK, DP), jnp.float32),
        pltpu.VMEM((CK, DP), jnp.float32),
        pltpu.VMEM((3, CK), jnp.int32),
        pltpu.VMEM((3, CK), jnp.int32),
        pltpu.VMEM_SHARED((NP, DP), jnp.float32),
        pltpu.SemaphoreType.DMA,
        pltpu.SemaphoreType.DMA,
        pltpu.SemaphoreType.DMA,
        pltpu.SemaphoreType.DMA,
        pltpu.SemaphoreType.DMA,
        pltpu.SemaphoreType.DMA,
    ],
    compiler_params=pltpu.CompilerParams(use_tc_tiling_on_sc=False),
)(_sc_agg_body)


def _tc_head_body(x_ref, agg_ref, wl_ref, bl_ref, wr_ref,
                  wlin_ref, blin_ref, emb_ref, log_ref):
    agg = agg_ref[0] + agg_ref[1]
    cnt = agg[:, D]
    mean = agg[:, :D] / jnp.maximum(cnt, 1.0)[:, None]
    dn = (((1,), (1,)), ((), ()))
    emb = (lax.dot_general(mean, wl_ref[...], dn,
                           preferred_element_type=jnp.float32)
           + lax.dot_general(x_ref[...], wr_ref[...], dn,
                             preferred_element_type=jnp.float32)
           + bl_ref[...])
    emb_ref[...] = emb
    h = jnp.maximum(emb, 0.0)
    log_ref[...] = lax.dot_general(h, wlin_ref[...], dn,
                                   preferred_element_type=jnp.float32) + blin_ref[...]


R = 512


def _tc_head(x, agg_p, W_l, b_l, W_r, W_lin, b_lin):
    n_cls = W_lin.shape[0]
    grid = NP // R
    return pl.pallas_call(
        _tc_head_body,
        grid=(grid,),
        in_specs=[
            pl.BlockSpec((R, D), lambda i: (i, 0)),
            pl.BlockSpec((NC, R, DP), lambda i: (0, i, 0)),
            pl.BlockSpec((D, D), lambda i: (0, 0)),
            pl.BlockSpec((1, D), lambda i: (0, 0)),
            pl.BlockSpec((D, D), lambda i: (0, 0)),
            pl.BlockSpec((n_cls, D), lambda i: (0, 0)),
            pl.BlockSpec((1, n_cls), lambda i: (0, 0)),
        ],
        out_specs=[
            pl.BlockSpec((R, D), lambda i: (i, 0)),
            pl.BlockSpec((R, n_cls), lambda i: (i, 0)),
        ],
        out_shape=[
            jax.ShapeDtypeStruct((N, D), jnp.float32),
            jax.ShapeDtypeStruct((N, n_cls), jnp.float32),
        ],
    )(x, agg_p, W_l, b_l, W_r, W_lin, b_lin)


def kernel(x, edge_index, W_l, b_l, W_r, W_lin, b_lin):
    ei = edge_index.astype(jnp.int32)
    pk = ei[0] | (ei[1] << 16)
    ones_col = jnp.ones((N, 1), jnp.float32)
    pad = jnp.zeros((N, DP - D - 1), jnp.float32)
    xp = jnp.concatenate([x, ones_col, pad], axis=1)
    agg_p = _sc_agg(xp, pk)
    emb, logits = _tc_head(x, agg_p, W_l, b_l.reshape(1, D),
                           W_r, W_lin, b_lin.reshape(1, -1))
    return (emb, logits)


# speedup vs baseline: 1.1958x; 1.1958x over previous
import functools

import jax
import jax.numpy as jnp
from jax import lax
from jax.experimental import pallas as pl
from jax.experimental.pallas import tpu as pltpu
from jax.experimental.pallas import tpu_sc as plsc

N = 10000
E = 320000
D = 128
NC = 2
NS = 16
NW = NC * NS
EPW = E // NW
CK = 80
NCHUNK = EPW // CK
NP = 10240
RPT = NP // NS

_MESH = plsc.VectorSubcoreMesh(
    core_axis_name="c", subcore_axis_name="s",
    num_cores=NC, num_subcores=NS)


def _sc_feat_body(x_hbm, pk_hbm, agg_out,
                  pk_t, gb0, gb1, gb2, srcb, dstb, accum,
                  semg0, semg1, semg2, sems0, sems1, sems2):
    c = lax.axis_index("c")
    s = lax.axis_index("s")
    wid = s * NC + c

    zeros16 = jnp.zeros((16,), jnp.float32)
    gbufs = (gb0, gb1, gb2)
    semgs = (semg0, semg1, semg2)
    semss = (sems0, sems1, sems2)

    def zero_gb0(i, carry):
        for k in range(D // 16):
            gb0[i, pl.ds(k * 16, 16)] = zeros16
        return carry
    lax.fori_loop(0, CK, zero_gb0, 0)

    base = s * RPT
    for k in range(RPT // CK):
        pltpu.sync_copy(gb0, accum.at[pl.ds(base + k * CK, CK)])

    pltpu.sync_copy(pk_hbm.at[pl.ds(wid * EPW, EPW)], pk_t)

    plsc.subcore_barrier()

    mask = jnp.full((16,), 0xFFFF, jnp.int32)

    def unpack(t, slot):
        for k in range(CK // 16):
            pkv = pk_t[pl.ds(t * CK + k * 16, 16)]
            srcb[slot, pl.ds(k * 16, 16)] = jnp.bitwise_and(pkv, mask)
            dstb[slot, pl.ds(k * 16, 16)] = lax.shift_right_logical(pkv, 16)

    def g_issue(t, b):
        pltpu.async_copy(x_hbm.at[srcb.at[b]], gbufs[b], semgs[b])

    def g_wait(b):
        pltpu.make_async_copy(x_hbm.at[srcb.at[b]], gbufs[b],
                              semgs[b]).wait()

    def s_issue(b):
        pltpu.async_copy(gbufs[b], accum.at[dstb.at[b]], semss[b], add=True)

    def s_wait(b):
        pltpu.make_async_copy(gbufs[b], accum.at[dstb.at[b]],
                              semss[b]).wait()

    unpack(0, 0)
    g_issue(0, 0)
    unpack(1, 1)
    g_issue(1, 1)
    g_wait(0); s_issue(0); unpack(2, 2); g_issue(2, 2)
    g_wait(1); s_issue(1); s_wait(0); unpack(3, 0); g_issue(3, 0)

    def triple(j, carry):
        t = 3 * j + 2
        for p, b in ((0, 2), (1, 0), (2, 1)):
            g_wait(b)
            s_issue(b)
            s_wait((b + 2) % 3)
            unpack(t + p + 2, (b + 2) % 3)
            g_issue(t + p + 2, (b + 2) % 3)
        return carry
    lax.fori_loop(0, 40, triple, 0)

    g_wait(2); s_issue(2); s_wait(1); unpack(124, 1); g_issue(124, 1)
    g_wait(0); s_issue(0); s_wait(2)
    g_wait(1); pltpu.sync_copy(gbufs[1], accum.at[dstb.at[1]], add=True)
    s_wait(0)

    plsc.subcore_barrier()

    pltpu.sync_copy(accum.at[pl.ds(base, RPT)],
                    agg_out.at[c, pl.ds(base, RPT)])


_sc_feat = functools.partial(
    pl.kernel,
    out_type=jax.ShapeDtypeStruct((NC, NP, D), jnp.float32),
    mesh=_MESH,
    scratch_types=[
        pltpu.VMEM((EPW,), jnp.int32),
        pltpu.VMEM((CK, D), jnp.float32),
        pltpu.VMEM((CK, D), jnp.float32),
        pltpu.VMEM((CK, D), jnp.float32),
        pltpu.VMEM((3, CK), jnp.int32),
        pltpu.VMEM((3, CK), jnp.int32),
        pltpu.VMEM_SHARED((NP, D), jnp.float32),
        pltpu.SemaphoreType.DMA,
        pltpu.SemaphoreType.DMA,
        pltpu.SemaphoreType.DMA,
        pltpu.SemaphoreType.DMA,
        pltpu.SemaphoreType.DMA,
        pltpu.SemaphoreType.DMA,
    ],
    compiler_params=pltpu.CompilerParams(use_tc_tiling_on_sc=True),
)(_sc_feat_body)

CW = 16


def _sc_cnt_body(pk_hbm, cnt_out,
                 pk_t, onesb, zb, dstb, accum, semc0, semc1, semc2):
    c = lax.axis_index("c")
    s = lax.axis_index("s")
    wid = s * NC + c

    zeros16 = jnp.zeros((16,), jnp.float32)
    one16 = jnp.where(lax.iota(jnp.int32, 16) == 0, 1.0, 0.0)
    semcs = (semc0, semc1, semc2)

    def zero_zb(i, carry):
        zb[i, pl.ds(0, 16)] = zeros16
        return carry
    lax.fori_loop(0, 128, zero_zb, 0)

    def init_ones(i, carry):
        onesb[i, pl.ds(0, 16)] = one16
        return carry
    lax.fori_loop(0, CK, init_ones, 0)

    base = s * RPT
    for k in range(RPT // 128):
        pltpu.sync_copy(zb, accum.at[pl.ds(base + k * 128, 128)])

    pltpu.sync_copy(pk_hbm.at[pl.ds(wid * EPW, EPW)], pk_t)

    plsc.subcore_barrier()

    def unpack(t, slot):
        for k in range(CK // 16):
            pkv = pk_t[pl.ds(t * CK + k * 16, 16)]
            dstb[slot, pl.ds(k * 16, 16)] = lax.shift_right_logical(pkv, 16)

    def s_issue(b):
        pltpu.async_copy(onesb, accum.at[dstb.at[b]], semcs[b], add=True)

    def s_wait(b):
        pltpu.make_async_copy(onesb, accum.at[dstb.at[b]],
                              semcs[b]).wait()

    unpack(0, 0)
    s_issue(0)
    unpack(1, 1)
    s_issue(1)
    s_wait(0)

    def triple(j, carry):
        t = 3 * j + 2
        for p, b in ((0, 2), (1, 0), (2, 1)):
            unpack(t + p, b)
            s_issue(b)
            s_wait((b + 2) % 3)
        return carry
    lax.fori_loop(0, 41, triple, 0)

    s_wait(1)

    plsc.subcore_barrier()

    pltpu.sync_copy(accum.at[pl.ds(base, RPT)],
                    cnt_out.at[c, pl.ds(base, RPT)])


_sc_cnt = functools.partial(
    pl.kernel,
    out_type=jax.ShapeDtypeStruct((NC, NP, CW), jnp.float32),
    mesh=_MESH,
    scratch_types=[
        pltpu.VMEM((EPW,), jnp.int32),
        pltpu.VMEM((CK, CW), jnp.float32),
        pltpu.VMEM((128, CW), jnp.float32),
        pltpu.VMEM((3, CK), jnp.int32),
        pltpu.VMEM_SHARED((NP, CW), jnp.float32),
        pltpu.SemaphoreType.DMA,
        pltpu.SemaphoreType.DMA,
        pltpu.SemaphoreType.DMA,
    ],
    compiler_params=pltpu.CompilerParams(use_tc_tiling_on_sc=False),
)(_sc_cnt_body)


def _tc_head_body(x_ref, agg_ref, cnt_ref, wl_ref, bl_ref, wr_ref,
                  wlin_ref, blin_ref, emb_ref, log_ref):
    agg = agg_ref[0] + agg_ref[1]
    cnt = cnt_ref[0, :, 0] + cnt_ref[1, :, 0]
    mean = agg / jnp.maximum(cnt, 1.0)[:, None]
    dn = (((1,), (1,)), ((), ()))
    emb = (lax.dot_general(mean, wl_ref[...], dn,
                           preferred_element_type=jnp.float32)
           + lax.dot_general(x_ref[...], wr_ref[...], dn,
                             preferred_element_type=jnp.float32)
           + bl_ref[...])
    emb_ref[...] = emb
    h = jnp.maximum(emb, 0.0)
    log_ref[...] = lax.dot_general(h, wlin_ref[...], dn,
                                   preferred_element_type=jnp.float32) + blin_ref[...]


R = 512


def _tc_head(x, agg_p, cnt_p, W_l, b_l, W_r, W_lin, b_lin):
    n_cls = W_lin.shape[0]
    grid = NP // R
    return pl.pallas_call(
        _tc_head_body,
        grid=(grid,),
        in_specs=[
            pl.BlockSpec((R, D), lambda i: (i, 0)),
            pl.BlockSpec((NC, R, D), lambda i: (0, i, 0)),
            pl.BlockSpec((NC, R, CW), lambda i: (0, i, 0)),
            pl.BlockSpec((D, D), lambda i: (0, 0)),
            pl.BlockSpec((1, D), lambda i: (0, 0)),
            pl.BlockSpec((D, D), lambda i: (0, 0)),
            pl.BlockSpec((n_cls, D), lambda i: (0, 0)),
            pl.BlockSpec((1, n_cls), lambda i: (0, 0)),
        ],
        out_specs=[
            pl.BlockSpec((R, D), lambda i: (i, 0)),
            pl.BlockSpec((R, n_cls), lambda i: (i, 0)),
        ],
        out_shape=[
            jax.ShapeDtypeStruct((N, D), jnp.float32),
            jax.ShapeDtypeStruct((N, n_cls), jnp.float32),
        ],
    )(x, agg_p, cnt_p, W_l, b_l, W_r, W_lin, b_lin)


def kernel(x, edge_index, W_l, b_l, W_r, W_lin, b_lin):
    ei = edge_index.astype(jnp.int32)
    pk = ei[0] | (ei[1] << 16)
    agg_p = _sc_feat(x, pk)
    cnt_p = _sc_cnt(pk)
    emb, logits = _tc_head(x, agg_p, cnt_p, W_l, b_l.reshape(1, D),
                           W_r, W_lin, b_lin.reshape(1, -1))
    return (emb, logits)


# speedup vs baseline: 1.2387x; 1.0359x over previous
import functools

import jax
import jax.numpy as jnp
from jax import lax
from jax.experimental import pallas as pl
from jax.experimental.pallas import tpu as pltpu
from jax.experimental.pallas import tpu_sc as plsc

N = 10000
E = 320000
D = 128
NC = 2
NS = 16
NW = NC * NS
EPW = E // NW
CK = 80
NCHUNK = EPW // CK
NP = 10240
RPT = NP // NS

_MESH = plsc.VectorSubcoreMesh(
    core_axis_name="c", subcore_axis_name="s",
    num_cores=NC, num_subcores=NS)


def _sc_feat_body(x_hbm, pk_hbm, agg_out,
                  pk_t, gb0, gb1, gb2, srcb, dstb, accum,
                  semg0, semg1, semg2, sems0, sems1, sems2):
    c = lax.axis_index("c")
    s = lax.axis_index("s")
    wid = s * NC + c

    zeros16 = jnp.zeros((16,), jnp.float32)
    gbufs = (gb0, gb1, gb2)
    semgs = (semg0, semg1, semg2)
    semss = (sems0, sems1, sems2)

    def zero_gb0(i, carry):
        for k in range(D // 16):
            gb0[i, pl.ds(k * 16, 16)] = zeros16
        return carry
    lax.fori_loop(0, CK, zero_gb0, 0)

    base = s * RPT
    for k in range(RPT // CK):
        pltpu.sync_copy(gb0, accum.at[pl.ds(base + k * CK, CK)])

    pltpu.sync_copy(pk_hbm.at[pl.ds(wid * EPW, EPW)], pk_t)

    plsc.subcore_barrier()

    mask = jnp.full((16,), 0xFFFF, jnp.int32)

    def unpack(t, slot):
        for k in range(CK // 16):
            pkv = pk_t[pl.ds(t * CK + k * 16, 16)]
            srcb[slot, pl.ds(k * 16, 16)] = jnp.bitwise_and(pkv, mask)
            dstb[slot, pl.ds(k * 16, 16)] = lax.shift_right_logical(pkv, 16)

    def g_issue(t, b):
        pltpu.async_copy(x_hbm.at[srcb.at[b]], gbufs[b], semgs[b])

    def g_wait(b):
        pltpu.make_async_copy(x_hbm.at[srcb.at[b]], gbufs[b],
                              semgs[b]).wait()

    def s_issue(b):
        pltpu.async_copy(gbufs[b], accum.at[dstb.at[b]], semss[b], add=True)

    def s_wait(b):
        pltpu.make_async_copy(gbufs[b], accum.at[dstb.at[b]],
                              semss[b]).wait()

    unpack(0, 0)
    g_issue(0, 0)
    unpack(1, 1)
    g_issue(1, 1)
    g_wait(0); s_issue(0); unpack(2, 2); g_issue(2, 2)
    g_wait(1); s_issue(1); s_wait(0); unpack(3, 0); g_issue(3, 0)

    def triple(j, carry):
        t = 3 * j + 2
        for p, b in ((0, 2), (1, 0), (2, 1)):
            g_wait(b)
            s_issue(b)
            s_wait((b + 2) % 3)
            unpack(t + p + 2, (b + 2) % 3)
            g_issue(t + p + 2, (b + 2) % 3)
        return carry
    lax.fori_loop(0, 40, triple, 0)

    g_wait(2); s_issue(2); s_wait(1); unpack(124, 1); g_issue(124, 1)
    g_wait(0); s_issue(0); s_wait(2)
    g_wait(1); pltpu.sync_copy(gbufs[1], accum.at[dstb.at[1]], add=True)
    s_wait(0)

    plsc.subcore_barrier()

    pltpu.sync_copy(accum.at[pl.ds(base, RPT)],
                    agg_out.at[c, pl.ds(base, RPT)])


_sc_feat = functools.partial(
    pl.kernel,
    out_type=jax.ShapeDtypeStruct((NC, NP, D), jnp.float32),
    mesh=_MESH,
    scratch_types=[
        pltpu.VMEM((EPW,), jnp.int32),
        pltpu.VMEM((CK, D), jnp.float32),
        pltpu.VMEM((CK, D), jnp.float32),
        pltpu.VMEM((CK, D), jnp.float32),
        pltpu.VMEM((3, CK), jnp.int32),
        pltpu.VMEM((3, CK), jnp.int32),
        pltpu.VMEM_SHARED((NP, D), jnp.float32),
        pltpu.SemaphoreType.DMA,
        pltpu.SemaphoreType.DMA,
        pltpu.SemaphoreType.DMA,
        pltpu.SemaphoreType.DMA,
        pltpu.SemaphoreType.DMA,
        pltpu.SemaphoreType.DMA,
    ],
    compiler_params=pltpu.CompilerParams(use_tc_tiling_on_sc=True),
)(_sc_feat_body)

CW = 16


def _sc_cnt_body(pk_hbm, cnt_out,
                 pk_t, onesb, zb, dstb, accum, semc0, semc1, semc2):
    c = lax.axis_index("c")
    s = lax.axis_index("s")
    wid = s * NC + c

    zeros16 = jnp.zeros((16,), jnp.float32)
    one16 = jnp.where(lax.iota(jnp.int32, 16) == 0, 1.0, 0.0)
    semcs = (semc0, semc1, semc2)

    def zero_zb(i, carry):
        zb[i, pl.ds(0, 16)] = zeros16
        return carry
    lax.fori_loop(0, 128, zero_zb, 0)

    def init_ones(i, carry):
        onesb[i, pl.ds(0, 16)] = one16
        return carry
    lax.fori_loop(0, CK, init_ones, 0)

    base = s * RPT
    for k in range(RPT // 128):
        pltpu.sync_copy(zb, accum.at[pl.ds(base + k * 128, 128)])

    pltpu.sync_copy(pk_hbm.at[pl.ds(wid * EPW, EPW)], pk_t)

    plsc.subcore_barrier()

    def unpack(t, slot):
        for k in range(CK // 16):
            pkv = pk_t[pl.ds(t * CK + k * 16, 16)]
            dstb[slot, pl.ds(k * 16, 16)] = lax.shift_right_logical(pkv, 16)

    def s_issue(b):
        pltpu.async_copy(onesb, accum.at[dstb.at[b]], semcs[b], add=True)

    def s_wait(b):
        pltpu.make_async_copy(onesb, accum.at[dstb.at[b]],
                              semcs[b]).wait()

    unpack(0, 0)
    s_issue(0)
    unpack(1, 1)
    s_issue(1)
    s_wait(0)

    def triple(j, carry):
        t = 3 * j + 2
        for p, b in ((0, 2), (1, 0), (2, 1)):
            unpack(t + p, b)
            s_issue(b)
            s_wait((b + 2) % 3)
        return carry
    lax.fori_loop(0, 41, triple, 0)

    s_wait(1)

    plsc.subcore_barrier()

    pltpu.sync_copy(accum.at[pl.ds(base, RPT)],
                    cnt_out.at[c, pl.ds(base, RPT)])


_sc_cnt = functools.partial(
    pl.kernel,
    out_type=jax.ShapeDtypeStruct((NC, NP, CW), jnp.float32),
    mesh=_MESH,
    scratch_types=[
        pltpu.VMEM((EPW,), jnp.int32),
        pltpu.VMEM((CK, CW), jnp.float32),
        pltpu.VMEM((128, CW), jnp.float32),
        pltpu.VMEM((3, CK), jnp.int32),
        pltpu.VMEM_SHARED((NP, CW), jnp.float32),
        pltpu.SemaphoreType.DMA,
        pltpu.SemaphoreType.DMA,
        pltpu.SemaphoreType.DMA,
    ],
    compiler_params=pltpu.CompilerParams(use_tc_tiling_on_sc=False),
)(_sc_cnt_body)


def _tc_head_body(x_ref, agg_ref, cnt_ref, wl_ref, bl_ref, wr_ref,
                  wlin_ref, blin_ref, emb_ref, log_ref):
    agg = agg_ref[0] + agg_ref[1]
    cnt = cnt_ref[0, :, 0] + cnt_ref[1, :, 0]
    mean = agg / jnp.maximum(cnt, 1.0)[:, None]
    dn = (((1,), (1,)), ((), ()))
    emb = (lax.dot_general(mean, wl_ref[...], dn,
                           preferred_element_type=jnp.float32)
           + lax.dot_general(x_ref[...], wr_ref[...], dn,
                             preferred_element_type=jnp.float32)
           + bl_ref[...])
    emb_ref[...] = emb
    h = jnp.maximum(emb, 0.0)
    log_ref[...] = lax.dot_general(h, wlin_ref[...], dn,
                                   preferred_element_type=jnp.float32) + blin_ref[...]


R = 1024


def _tc_head(x, agg_p, cnt_p, W_l, b_l, W_r, W_lin, b_lin):
    n_cls = W_lin.shape[0]
    grid = NP // R
    return pl.pallas_call(
        _tc_head_body,
        grid=(grid,),
        in_specs=[
            pl.BlockSpec((R, D), lambda i: (i, 0)),
            pl.BlockSpec((NC, R, D), lambda i: (0, i, 0)),
            pl.BlockSpec((NC, R, CW), lambda i: (0, i, 0)),
            pl.BlockSpec((D, D), lambda i: (0, 0)),
            pl.BlockSpec((1, D), lambda i: (0, 0)),
            pl.BlockSpec((D, D), lambda i: (0, 0)),
            pl.BlockSpec((n_cls, D), lambda i: (0, 0)),
            pl.BlockSpec((1, n_cls), lambda i: (0, 0)),
        ],
        out_specs=[
            pl.BlockSpec((R, D), lambda i: (i, 0)),
            pl.BlockSpec((R, n_cls), lambda i: (i, 0)),
        ],
        out_shape=[
            jax.ShapeDtypeStruct((N, D), jnp.float32),
            jax.ShapeDtypeStruct((N, n_cls), jnp.float32),
        ],
    )(x, agg_p, cnt_p, W_l, b_l, W_r, W_lin, b_lin)


def kernel(x, edge_index, W_l, b_l, W_r, W_lin, b_lin):
    ei = edge_index.astype(jnp.int32)
    pk = ei[0] | (ei[1] << 16)
    agg_p = _sc_feat(x, pk)
    cnt_p = _sc_cnt(pk)
    emb, logits = _tc_head(x, agg_p, cnt_p, W_l, b_l.reshape(1, D),
                           W_r, W_lin, b_lin.reshape(1, -1))
    return (emb, logits)
